# baseline (speedup 1.0000x reference)
import functools

import jax
import jax.numpy as jnp
from jax import lax
from jax.experimental import pallas as pl
from jax.experimental.pallas import tpu as pltpu
from jax.experimental.pallas import tpu_sc as plsc

B = 4096 * 200
D = 64
NC, NS = 2, 16
NW = NC * NS
BPW = B // NW
CH = 1024
NCHUNK = BPW // CH

_MESH = plsc.VectorSubcoreMesh(core_axis_name="c", subcore_axis_name="s")


@functools.partial(
    pl.kernel,
    mesh=_MESH,
    out_type=jax.ShapeDtypeStruct((B, D), jnp.float32),
    scratch_types=[
        pltpu.VMEM((CH,), jnp.int32),
        pltpu.VMEM((CH, D), jnp.float32),
        pltpu.SemaphoreType.DMA,
    ],
    compiler_params=pltpu.CompilerParams(use_tc_tiling_on_sc=False),
)
def _gather_kernel(idx_hbm, table_hbm, out_hbm, idx_v, rows_v, sem):
    wid = lax.axis_index("s") * NC + lax.axis_index("c")
    base = wid * BPW

    def chunk(g, carry):
        off = base + g * CH
        pltpu.sync_copy(idx_hbm.at[pl.ds(off, CH)], idx_v)
        pltpu.async_copy(table_hbm.at[idx_v], rows_v, sem).wait()
        pltpu.sync_copy(rows_v, out_hbm.at[pl.ds(off, CH)])
        return carry

    lax.fori_loop(0, NCHUNK, chunk, 0)


def kernel(indices, table):
    flat_idx = indices.reshape(-1)
    out = _gather_kernel(flat_idx, table)
    return out.reshape(indices.shape + (table.shape[1],))


# speedup vs baseline: 1.0169x; 1.0169x over previous
import functools

import jax
import jax.numpy as jnp
from jax import lax
from jax.experimental import pallas as pl
from jax.experimental.pallas import tpu as pltpu
from jax.experimental.pallas import tpu_sc as plsc

B = 4096 * 200
D = 64
NC, NS = 2, 16
NW = NC * NS
BPW = B // NW
CH = 400
NBUF = 4
LOOKAHEAD = 2
NCHUNK = BPW // CH
NSTEP = NCHUNK // NBUF

_MESH = plsc.VectorSubcoreMesh(core_axis_name="c", subcore_axis_name="s")


@functools.partial(
    pl.kernel,
    mesh=_MESH,
    out_type=jax.ShapeDtypeStruct((B, D), jnp.float32),
    scratch_types=[
        pltpu.VMEM((BPW,), jnp.int32),
        pltpu.VMEM((NBUF, CH, D), jnp.float32),
        pltpu.SemaphoreType.DMA((NBUF,)),
        pltpu.SemaphoreType.DMA((NBUF,)),
    ],
    compiler_params=pltpu.CompilerParams(use_tc_tiling_on_sc=False),
)
def _gather_kernel(idx_hbm, table_hbm, out_hbm, idx_v, rows_v, sg, so):
    wid = lax.axis_index("s") * NC + lax.axis_index("c")
    base = wid * BPW

    def gather(g, b):
        return pltpu.make_async_copy(
            table_hbm.at[idx_v.at[pl.ds(g * CH, CH)]], rows_v.at[b], sg.at[b])

    def store(g, b):
        return pltpu.make_async_copy(
            rows_v.at[b], out_hbm.at[pl.ds(base + g * CH, CH)], so.at[b])

    pltpu.sync_copy(idx_hbm.at[pl.ds(base, BPW)], idx_v)
    for j in range(LOOKAHEAD):
        gather(j, j).start()

    def step(t_step, carry):
        for j in range(NBUF):
            t = t_step * NBUF + j
            gather(t, j).wait()
            bl = (j + LOOKAHEAD) % NBUF

            @pl.when(t >= NBUF - LOOKAHEAD)
            def _wait_store():
                store(0, bl).wait()

            @pl.when(t + LOOKAHEAD < NCHUNK)
            def _next_gather():
                gather(t + LOOKAHEAD, bl).start()

            store(t, j).start()
        return carry

    lax.fori_loop(0, NSTEP, step, 0)
    store(0, (NCHUNK - 2) % NBUF).wait()
    store(0, (NCHUNK - 1) % NBUF).wait()


def kernel(indices, table):
    flat_idx = indices.reshape(-1)
    out = _gather_kernel(flat_idx, table)
    return out.reshape(indices.shape + (table.shape[1],))


# speedup vs baseline: 1.0198x; 1.0029x over previous
import functools

import jax
import jax.numpy as jnp
from jax import lax
from jax.experimental import pallas as pl
from jax.experimental.pallas import tpu as pltpu
from jax.experimental.pallas import tpu_sc as plsc

BATCH = 4096
HIST = 200
D = 64
NC, NS = 2, 16
NW = NC * NS
RPW = BATCH // NW
NBUF = 4
LOOKAHEAD = 2
NSTEP = RPW // NBUF

_MESH = plsc.VectorSubcoreMesh(core_axis_name="c", subcore_axis_name="s")


@functools.partial(
    pl.kernel,
    mesh=_MESH,
    out_type=jax.ShapeDtypeStruct((BATCH, HIST, D), jnp.float32),
    scratch_types=[
        pltpu.VMEM((RPW, HIST), jnp.int32),
        pltpu.VMEM((NBUF, HIST, D), jnp.float32),
        pltpu.SemaphoreType.DMA((NBUF,)),
        pltpu.SemaphoreType.DMA((NBUF,)),
    ],
    compiler_params=pltpu.CompilerParams(use_tc_tiling_on_sc=False),
)
def _gather_kernel(idx_hbm, table_hbm, out_hbm, idx_v, rows_v, sg, so):
    wid = lax.axis_index("s") * NC + lax.axis_index("c")
    base = wid * RPW

    def gather(i, b):
        return pltpu.make_async_copy(
            table_hbm.at[idx_v.at[i]], rows_v.at[b], sg.at[b])

    def store(i, b):
        return pltpu.make_async_copy(
            rows_v.at[b], out_hbm.at[base + i], so.at[b])

    pltpu.sync_copy(idx_hbm.at[pl.ds(base, RPW), :], idx_v)
    for j in range(LOOKAHEAD):
        gather(j, j).start()

    def step(t_step, carry):
        for j in range(NBUF):
            t = t_step * NBUF + j
            gather(t, j).wait()
            bl = (j + LOOKAHEAD) % NBUF

            @pl.when(t >= NBUF - LOOKAHEAD)
            def _wait_store():
                store(0, bl).wait()

            @pl.when(t + LOOKAHEAD < RPW)
            def _next_gather():
                gather(t + LOOKAHEAD, bl).start()

            store(t, j).start()
        return carry

    lax.fori_loop(0, NSTEP, step, 0)
    store(0, (RPW - 2) % NBUF).wait()
    store(0, (RPW - 1) % NBUF).wait()


def kernel(indices, table):
    return _gather_kernel(indices, table)
